# Initial kernel scaffold; baseline (speedup 1.0000x reference)
import jax
import jax.numpy as jnp
from jax.experimental import pallas as pl


def kernel(x, edge_index, ln_g, ln_b, enc_w1, enc_b1, enc_w2, enc_b2, gcn1_w, gcn1_b, bn1_g, bn1_b, gat_w, gat_as, gat_ad, gat_b, bn2_g, bn2_b, gcn2_w, gcn2_b, bn3_g, bn3_b, pr_w1, pr_b1, pr_w2, pr_b2):
    raise NotImplementedError("write your pallas kernel here")


# speedup vs baseline: 1.2565x; 1.2565x over previous
import functools

import jax
import jax.numpy as jnp
from jax.experimental import pallas as pl

N = 100000
E = 1600000
HID = 32
HEADS = 2


def _bn(x, g, b, eps=1e-5):
    m = jnp.mean(x, axis=0)
    v = jnp.var(x, axis=0)
    return (x - m) / jnp.sqrt(v + eps) * g + b


def _final_mlp_block(h_ref, w1_ref, b1_ref, w2_ref, b2_ref, o_ref):
    h = h_ref[...]
    h = jnp.maximum(jnp.dot(h, w1_ref[...], preferred_element_type=jnp.float32)
                    + b1_ref[...], 0.0)
    o = jnp.dot(h, w2_ref[...], preferred_element_type=jnp.float32) + b2_ref[...]
    o_ref[...] = jax.nn.sigmoid(o.reshape(o_ref.shape))


def _final_mlp(h3, pr_w1, pr_b1, pr_w2, pr_b2):
    out = pl.pallas_call(
        _final_mlp_block,
        out_shape=jax.ShapeDtypeStruct((N // 1000, 1000), jnp.float32),
    )(h3, pr_w1, pr_b1, pr_w2, pr_b2)
    return out.reshape(N)


def kernel(x, edge_index, ln_g, ln_b, enc_w1, enc_b1, enc_w2, enc_b2, gcn1_w, gcn1_b, bn1_g, bn1_b, gat_w, gat_as, gat_ad, gat_b, bn2_g, bn2_b, gcn2_w, gcn2_b, bn3_g, bn3_b, pr_w1, pr_b1, pr_w2, pr_b2):
    src = edge_index[0]
    dst = edge_index[1]

    m = jnp.mean(x, axis=-1, keepdims=True)
    v = jnp.var(x, axis=-1, keepdims=True)
    h = (x - m) / jnp.sqrt(v + 1e-5) * ln_g + ln_b
    h = jax.nn.relu(h @ enc_w1 + enc_b1)
    h = jax.nn.relu(h @ enc_w2 + enc_b2)

    deg = jax.ops.segment_sum(jnp.ones((E,), jnp.float32), dst, num_segments=N) + 1.0
    dis = jax.lax.rsqrt(deg)

    g = (h @ gcn1_w) * dis[:, None]
    p1 = jax.ops.segment_sum(g[src], dst, num_segments=N) + g
    h = jax.nn.relu(_bn(dis[:, None] * p1 + gcn1_b, bn1_g, bn1_b))

    C = HID // HEADS
    hh = (h @ gat_w).reshape(N, HEADS, C)
    as_ = jnp.sum(hh * gat_as[None], axis=-1)
    ad_ = jnp.sum(hh * gat_ad[None], axis=-1)
    bound = jnp.max(as_, axis=0) + jnp.max(ad_, axis=0)
    M = jnp.where(bound > 0, bound, 0.2 * bound)
    e = as_[src] + ad_[dst]
    e = jnp.where(e > 0, e, 0.2 * e) - M
    w = jnp.exp(e)
    eself = as_ + ad_
    wself = jnp.exp(jnp.where(eself > 0, eself, 0.2 * eself) - M)
    den = jax.ops.segment_sum(w, dst, num_segments=N) + wself
    numer = jax.ops.segment_sum(hh[src] * w[:, :, None], dst, num_segments=N) \
        + hh * wself[:, :, None]
    h = (numer / (den[:, :, None] + 1e-16)).reshape(N, HID) + gat_b
    h = jax.nn.relu(_bn(h, bn2_g, bn2_b))

    f = (h @ gcn2_w) * dis[:, None]
    p2 = jax.ops.segment_sum(f[src], dst, num_segments=N) + f
    h3 = jax.nn.relu(_bn(dis[:, None] * p2 + gcn2_b, bn3_g, bn3_b))

    return _final_mlp(h3, pr_w1, pr_b1, pr_w2, pr_b2)


# speedup vs baseline: 60.5744x; 48.2069x over previous
import jax
import jax.numpy as jnp
from jax import lax
from jax.experimental import pallas as pl
from jax.experimental.pallas import tpu as pltpu
from jax.experimental.pallas import tpu_sc as plsc

N = 100000
E = 1600000
HID = 32
HEADS = 2

NC = 2
NS = 16
LANES = 16
NW = NC * NS

WROWS = 392
EROWS = NW * WROWS
EPAD = EROWS * 128 - E

ZROWS = 6256
LASTZ = N - (NS - 1) * ZROWS

_MESH = plsc.VectorSubcoreMesh(core_axis_name="c", subcore_axis_name="s")


def _fill(ref, start, nvec, value):
    def body(i, _):
        ref[pl.ds(start + i * LANES, LANES)] = jnp.full((LANES,), value, ref.dtype)
        return 0

    lax.fori_loop(0, nvec, body, 0)


def _deg_body(dst2d, p0, p1, acc, idxbig, ones_row, zbuf, sem):
    cid = lax.axis_index("c")
    sid = lax.axis_index("s")
    wid = sid * NC + cid

    _fill(zbuf, 0, ZROWS // LANES, 0.0)
    _fill(ones_row, 0, 128 // LANES, 1.0)
    base = sid * ZROWS

    @pl.when(sid < NS - 1)
    def _():
        pltpu.sync_copy(zbuf.at[pl.ds(0, ZROWS)], acc.at[pl.ds(base, ZROWS)])

    @pl.when(sid == NS - 1)
    def _():
        pltpu.sync_copy(zbuf.at[pl.ds(0, LASTZ)], acc.at[pl.ds(base, LASTZ)])

    plsc.subcore_barrier()

    row0 = wid * WROWS
    pltpu.sync_copy(dst2d.at[pl.ds(row0, WROWS)], idxbig)

    def srow(j, _):
        pltpu.sync_copy(ones_row, acc.at[idxbig.at[j]], add=True)
        return 0

    lax.fori_loop(0, WROWS, srow, 0)
    plsc.subcore_barrier()

    @pl.when(sid < NS - 1)
    def _():
        pltpu.sync_copy(acc.at[pl.ds(base, ZROWS)], zbuf.at[pl.ds(0, ZROWS)])

    @pl.when(sid == NS - 1)
    def _():
        pltpu.sync_copy(acc.at[pl.ds(base, LASTZ)], zbuf.at[pl.ds(0, LASTZ)])

    @pl.when(cid == 0)
    def _():
        @pl.when(sid < NS - 1)
        def _():
            pltpu.sync_copy(zbuf.at[pl.ds(0, ZROWS)], p0.at[pl.ds(base, ZROWS)])

        @pl.when(sid == NS - 1)
        def _():
            pltpu.sync_copy(zbuf.at[pl.ds(0, LASTZ)], p0.at[pl.ds(base, LASTZ)])

    @pl.when(cid == 1)
    def _():
        @pl.when(sid < NS - 1)
        def _():
            pltpu.sync_copy(zbuf.at[pl.ds(0, ZROWS)], p1.at[pl.ds(base, ZROWS)])

        @pl.when(sid == NS - 1)
        def _():
            pltpu.sync_copy(zbuf.at[pl.ds(0, LASTZ)], p1.at[pl.ds(base, LASTZ)])


_deg_call = pl.kernel(
    _deg_body,
    out_type=[jax.ShapeDtypeStruct((N,), jnp.float32) for _ in range(2)],
    mesh=_MESH,
    scratch_types=[
        pltpu.VMEM_SHARED((N + 8,), jnp.float32),
        pltpu.VMEM((WROWS, 128), jnp.int32),
        pltpu.VMEM((128,), jnp.float32),
        pltpu.VMEM((ZROWS,), jnp.float32),
        pltpu.SemaphoreType.DMA,
    ],
)


_FCH = 784


def _row_chunks(total):
    full, rem = divmod(total, _FCH)
    out = [(i * _FCH, _FCH) for i in range(full)]
    if rem:
        out.append((full * _FCH, rem))
    return out


def _zero16(acc, zbuf, sid):
    def zrow(i, _):
        zbuf[i] = jnp.zeros((LANES,), jnp.float32)
        return 0

    lax.fori_loop(0, _FCH, zrow, 0)

    def do(total):
        for off, sz in _row_chunks(total):
            pltpu.sync_copy(zbuf.at[pl.ds(0, sz)],
                            acc.at[pl.ds(sid * ZROWS + off, sz)])

    @pl.when(sid < NS - 1)
    def _():
        do(ZROWS)

    @pl.when(sid == NS - 1)
    def _():
        do(LASTZ)


def _flush16(acc, out, zbuf, sid):
    def do(total):
        for off, sz in _row_chunks(total):
            pltpu.sync_copy(acc.at[pl.ds(sid * ZROWS + off, sz)],
                            zbuf.at[pl.ds(0, sz)])
            pltpu.sync_copy(zbuf.at[pl.ds(0, sz)],
                            out.at[pl.ds(sid * ZROWS + off, sz)])

    @pl.when(sid < NS - 1)
    def _():
        do(ZROWS)

    @pl.when(sid == NS - 1)
    def _():
        do(LASTZ)


def _zero1(acc, zbuf, sid):
    _fill(zbuf, 0, ZROWS // LANES, 0.0)
    base = sid * ZROWS

    @pl.when(sid < NS - 1)
    def _():
        pltpu.sync_copy(zbuf.at[pl.ds(0, ZROWS)], acc.at[pl.ds(base, ZROWS)])

    @pl.when(sid == NS - 1)
    def _():
        pltpu.sync_copy(zbuf.at[pl.ds(0, LASTZ)], acc.at[pl.ds(base, LASTZ)])


def _flush1(acc, out, zbuf, sid):
    base = sid * ZROWS

    @pl.when(sid < NS - 1)
    def _():
        pltpu.sync_copy(acc.at[pl.ds(base, ZROWS)], zbuf.at[pl.ds(0, ZROWS)])
        pltpu.sync_copy(zbuf.at[pl.ds(0, ZROWS)], out.at[pl.ds(base, ZROWS)])

    @pl.when(sid == NS - 1)
    def _():
        pltpu.sync_copy(acc.at[pl.ds(base, LASTZ)], zbuf.at[pl.ds(0, LASTZ)])
        pltpu.sync_copy(zbuf.at[pl.ds(0, LASTZ)], out.at[pl.ds(base, LASTZ)])


_SCH = 8


def _make_segsum(rows_per_tile, edge_split):
    nchunks = rows_per_tile // _SCH

    def body(src2d, dst2d, g0, g1, o0, o1, acc, sidx, didx, rows, zbuf, sem):
        cid = lax.axis_index("c")
        sid = lax.axis_index("s")
        _zero16(acc, zbuf, sid)
        plsc.subcore_barrier()

        if edge_split:
            row0 = cid * (EROWS // 2) + sid * rows_per_tile
        else:
            row0 = sid * rows_per_tile

        def inner(gtab):
            def chunk(ci, _):
                r = row0 + ci * _SCH
                pltpu.sync_copy(src2d.at[pl.ds(r, _SCH)], sidx)
                pltpu.sync_copy(dst2d.at[pl.ds(r, _SCH)], didx)
                descs = [pltpu.async_copy(gtab.at[sidx.at[j]], rows.at[j], sem)
                         for j in range(_SCH)]
                for d in descs:
                    d.wait()
                for j in range(_SCH):
                    pltpu.sync_copy(rows.at[j], acc.at[didx.at[j]], add=True)
                return 0

            lax.fori_loop(0, nchunks, chunk, 0)

        @pl.when(cid == 0)
        def _():
            inner(g0)

        @pl.when(cid == 1)
        def _():
            inner(g1)

        plsc.subcore_barrier()

        @pl.when(cid == 0)
        def _():
            _flush16(acc, o0, zbuf, sid)

        @pl.when(cid == 1)
        def _():
            _flush16(acc, o1, zbuf, sid)

    return pl.kernel(
        body,
        out_type=[jax.ShapeDtypeStruct((N, 16), jnp.float32) for _ in range(2)],
        mesh=_MESH,
        scratch_types=[
            pltpu.VMEM_SHARED((N + 8, 16), jnp.float32),
            pltpu.VMEM((_SCH, 128), jnp.int32),
            pltpu.VMEM((_SCH, 128), jnp.int32),
            pltpu.VMEM((_SCH, 128, 16), jnp.float32),
            pltpu.VMEM((_FCH, 16), jnp.float32),
            pltpu.SemaphoreType.DMA,
        ],
        compiler_params=pltpu.CompilerParams(use_tc_tiling_on_sc=False, needs_layout_passes=False),
    )


_segsum_chsplit = _make_segsum(EROWS // NS, False)
_segsum_esplit = _make_segsum(EROWS // NW, True)


_GCH = 4


def _gat_body(src2d, dst2d, hh0, hh1, as0, as1, ad0, ad1, mrows,
              on0, on1, wout,
              accn, sidx, didx, asb, adb, hhb, wc, mvb, zbuf, sem):
    cid = lax.axis_index("c")
    sid = lax.axis_index("s")
    _zero16(accn, zbuf, sid)
    pltpu.sync_copy(mrows.at[pl.ds(cid, 1)], mvb)
    plsc.subcore_barrier()

    iota = lax.iota(jnp.int32, LANES)
    mv = mvb[0]
    row0 = sid * (EROWS // NS)
    nchunks = (EROWS // NS) // _GCH

    def inner(hhtab, astab, adtab, wbase):
        def chunk(ci, _):
            r = row0 + ci * _GCH
            pltpu.sync_copy(src2d.at[pl.ds(r, _GCH)], sidx)
            pltpu.sync_copy(dst2d.at[pl.ds(r, _GCH)], didx)
            descs = []
            for j in range(_GCH):
                descs.append(pltpu.async_copy(astab.at[sidx.at[j]], asb.at[j], sem))
                descs.append(pltpu.async_copy(adtab.at[didx.at[j]], adb.at[j], sem))
                descs.append(pltpu.async_copy(hhtab.at[sidx.at[j]], hhb.at[j], sem))
            for d in descs:
                d.wait()

            for j in range(_GCH):
                def grp(gidx, _):
                    e0 = gidx * LANES
                    ev = asb[j, pl.ds(e0, LANES)] + adb[j, pl.ds(e0, LANES)]
                    ev = jnp.where(ev > 0.0, ev, 0.2 * ev) - mv
                    wv = jnp.exp(ev)
                    wc[j, pl.ds(e0, LANES)] = wv
                    idx_e = iota + e0
                    for k in range(16):
                        idx_k = jnp.full((LANES,), k, jnp.int32)
                        hv = plsc.load_gather(hhb.at[j], [idx_e, idx_k])
                        plsc.store_scatter(hhb.at[j], [idx_e, idx_k], hv * wv)
                    return 0

                lax.fori_loop(0, 128 // LANES, grp, 0)

            pltpu.sync_copy(wc, wout.at[pl.ds(wbase + r, _GCH)])
            for j in range(_GCH):
                pltpu.sync_copy(hhb.at[j], accn.at[didx.at[j]], add=True)
            return 0

        lax.fori_loop(0, nchunks, chunk, 0)

    @pl.when(cid == 0)
    def _():
        inner(hh0, as0, ad0, 0)

    @pl.when(cid == 1)
    def _():
        inner(hh1, as1, ad1, EROWS)

    plsc.subcore_barrier()

    @pl.when(cid == 0)
    def _():
        _flush16(accn, on0, zbuf, sid)

    @pl.when(cid == 1)
    def _():
        _flush16(accn, on1, zbuf, sid)


_gat_call = pl.kernel(
    _gat_body,
    out_type=[jax.ShapeDtypeStruct((N, 16), jnp.float32),
              jax.ShapeDtypeStruct((N, 16), jnp.float32),
              jax.ShapeDtypeStruct((2 * EROWS, 128), jnp.float32)],
    mesh=_MESH,
    scratch_types=[
        pltpu.VMEM_SHARED((N + 8, 16), jnp.float32),
        pltpu.VMEM((_GCH, 128), jnp.int32),
        pltpu.VMEM((_GCH, 128), jnp.int32),
        pltpu.VMEM((_GCH, 128), jnp.float32),
        pltpu.VMEM((_GCH, 128), jnp.float32),
        pltpu.VMEM((_GCH, 128, 16), jnp.float32),
        pltpu.VMEM((_GCH, 128), jnp.float32),
        pltpu.VMEM((1, 16), jnp.float32),
        pltpu.VMEM((_FCH, 16), jnp.float32),
        pltpu.SemaphoreType.DMA,
    ],
    compiler_params=pltpu.CompilerParams(use_tc_tiling_on_sc=False, needs_layout_passes=False),
)


_DCH = 8


def _wden_body(w2d, dst2d, d0, d1, acc, widx, wval, zbuf, sem):
    cid = lax.axis_index("c")
    sid = lax.axis_index("s")
    _zero1(acc, zbuf, sid)
    plsc.subcore_barrier()

    row0 = sid * (EROWS // NS)
    nchunks = (EROWS // NS) // _DCH
    wbase = cid * EROWS

    def chunk(ci, _):
        r = row0 + ci * _DCH
        pltpu.sync_copy(dst2d.at[pl.ds(r, _DCH)], widx)
        pltpu.sync_copy(w2d.at[pl.ds(wbase + r, _DCH)], wval)
        for j in range(_DCH):
            pltpu.sync_copy(wval.at[j], acc.at[widx.at[j]], add=True)
        return 0

    lax.fori_loop(0, nchunks, chunk, 0)
    plsc.subcore_barrier()

    @pl.when(cid == 0)
    def _():
        _flush1(acc, d0, zbuf, sid)

    @pl.when(cid == 1)
    def _():
        _flush1(acc, d1, zbuf, sid)


_wden_call = pl.kernel(
    _wden_body,
    out_type=[jax.ShapeDtypeStruct((N,), jnp.float32) for _ in range(2)],
    mesh=_MESH,
    scratch_types=[
        pltpu.VMEM_SHARED((N + 8,), jnp.float32),
        pltpu.VMEM((_DCH, 128), jnp.int32),
        pltpu.VMEM((_DCH, 128), jnp.float32),
        pltpu.VMEM((ZROWS,), jnp.float32),
        pltpu.SemaphoreType.DMA,
    ],
    compiler_params=pltpu.CompilerParams(use_tc_tiling_on_sc=False, needs_layout_passes=False),
)


B = 2000
GRID = N // B


def _rb(width):
    return pl.BlockSpec((B, width), lambda i: (i, 0))


def _full(shape):
    return pl.BlockSpec(shape, lambda i: tuple(0 for _ in shape))


def _enc_block(x_ref, lng, lnb, w1, b1, w2, b2, o_ref):
    x = x_ref[...]
    m = jnp.mean(x, axis=1, keepdims=True)
    v = jnp.mean((x - m) ** 2, axis=1, keepdims=True)
    h = (x - m) * lax.rsqrt(v + 1e-5) * lng[...] + lnb[...]
    h = jnp.maximum(jnp.dot(h, w1[...], preferred_element_type=jnp.float32) + b1[...], 0.0)
    h = jnp.maximum(jnp.dot(h, w2[...], preferred_element_type=jnp.float32) + b2[...], 0.0)
    o_ref[...] = h


def enc(x, lng, lnb, w1, b1, w2, b2):
    return pl.pallas_call(
        _enc_block, grid=(GRID,),
        in_specs=[_rb(6), _full((6,)), _full((6,)), _full((6, 32)),
                  _full((32,)), _full((32, 32)), _full((32,))],
        out_specs=_rb(32),
        out_shape=jax.ShapeDtypeStruct((N, 32), jnp.float32),
    )(x, lng, lnb, w1, b1, w2, b2)


def _g_block(p0, p1, h0, w, o_dis, o_g0, o_g1):
    deg = p0[...] + p1[...] + 1.0
    dis = lax.rsqrt(deg)
    g = jnp.dot(h0[...], w[...], preferred_element_type=jnp.float32) * dis
    o_dis[...] = dis
    o_g0[...] = g[:, :16]
    o_g1[...] = g[:, 16:]


def gprep(p0, p1, h0, w):
    return pl.pallas_call(
        _g_block, grid=(GRID,),
        in_specs=[_rb(1), _rb(1), _rb(32), _full((32, 32))],
        out_specs=[_rb(1), _rb(16), _rb(16)],
        out_shape=[jax.ShapeDtypeStruct((N, 1), jnp.float32),
                   jax.ShapeDtypeStruct((N, 16), jnp.float32),
                   jax.ShapeDtypeStruct((N, 16), jnp.float32)],
    )(p0, p1, h0, w)


def zstat16x2(s0, s1, g0, g1, dis, bias):
    def blk(s0r, s1r, g0r, g1r, disr, br, o_z, o_st):
        d = disr[...]
        z = jnp.concatenate([d * (s0r[...] + g0r[...]),
                             d * (s1r[...] + g1r[...])], axis=1) + br[...]
        o_z[...] = z

        @pl.when(pl.program_id(0) == 0)
        def _():
            o_st[...] = jnp.zeros_like(o_st)

        o_st[...] += jnp.stack([jnp.sum(z, axis=0), jnp.sum(z * z, axis=0)])

    return pl.pallas_call(
        blk, grid=(GRID,),
        in_specs=[_rb(16), _rb(16), _rb(16), _rb(16), _rb(1), _full((32,))],
        out_specs=[_rb(32), _full((2, 32))],
        out_shape=[jax.ShapeDtypeStruct((N, 32), jnp.float32),
                   jax.ShapeDtypeStruct((2, 32), jnp.float32)],
    )(s0, s1, g0, g1, dis, bias)


def bn1_gat_prep(z, st, bng, bnb, gat_w, gat_as, gat_ad):
    def blk(zr, str_, bngr, bnbr, wr, asr, adr, o_h0, o_h1, o_sa, o_mx):
        mean = str_[0] / N
        var = str_[1] / N - mean * mean
        h = (zr[...] - mean) * lax.rsqrt(var + 1e-5) * bngr[...] + bnbr[...]
        h = jnp.maximum(h, 0.0)
        hh = jnp.dot(h, wr[...], preferred_element_type=jnp.float32)
        hh0 = hh[:, :16]
        hh1 = hh[:, 16:]
        a0 = jnp.sum(hh0 * asr[0], axis=1)
        a1 = jnp.sum(hh1 * asr[1], axis=1)
        d0 = jnp.sum(hh0 * adr[0], axis=1)
        d1 = jnp.sum(hh1 * adr[1], axis=1)
        o_h0[...] = hh0
        o_h1[...] = hh1
        o_sa[...] = jnp.stack([a0, a1, d0, d1], axis=1)

        @pl.when(pl.program_id(0) == 0)
        def _():
            o_mx[...] = jnp.full_like(o_mx, -3.0e38)

        mx = jnp.stack([jnp.max(a0), jnp.max(a1), jnp.max(d0), jnp.max(d1)])
        o_mx[...] = jnp.maximum(o_mx[...], mx[None, :])

    return pl.pallas_call(
        blk, grid=(GRID,),
        in_specs=[_rb(32), _full((2, 32)), _full((32,)), _full((32,)),
                  _full((32, 32)), _full((2, 16)), _full((2, 16))],
        out_specs=[_rb(16), _rb(16), _rb(4), _full((1, 4))],
        out_shape=[jax.ShapeDtypeStruct((N, 16), jnp.float32),
                   jax.ShapeDtypeStruct((N, 16), jnp.float32),
                   jax.ShapeDtypeStruct((N, 4), jnp.float32),
                   jax.ShapeDtypeStruct((1, 4), jnp.float32)],
    )(z, st, bng, bnb, gat_w, gat_as, gat_ad)


def gat_epilogue(on0, on1, od0, od1, hh0, hh1, sa, mrow, gat_b):
    def blk(on0r, on1r, od0r, od1r, h0r, h1r, sar, mr, br, o_z, o_st):
        es = sar[..., 0:2] + sar[..., 2:4]
        ws = jnp.exp(jnp.where(es > 0, es, 0.2 * es) - mr[...])
        n0 = on0r[...] + h0r[...] * ws[:, 0:1]
        n1 = on1r[...] + h1r[...] * ws[:, 1:2]
        d0 = od0r[...] + ws[:, 0:1]
        d1 = od1r[...] + ws[:, 1:2]
        z = jnp.concatenate([n0 / (d0 + 1e-16), n1 / (d1 + 1e-16)], axis=1) + br[...]
        o_z[...] = z

        @pl.when(pl.program_id(0) == 0)
        def _():
            o_st[...] = jnp.zeros_like(o_st)

        o_st[...] += jnp.stack([jnp.sum(z, axis=0), jnp.sum(z * z, axis=0)])

    return pl.pallas_call(
        blk, grid=(GRID,),
        in_specs=[_rb(16), _rb(16), _rb(1), _rb(1), _rb(16), _rb(16),
                  _rb(4), _full((1, 2)), _full((32,))],
        out_specs=[_rb(32), _full((2, 32))],
        out_shape=[jax.ShapeDtypeStruct((N, 32), jnp.float32),
                   jax.ShapeDtypeStruct((2, 32), jnp.float32)],
    )(on0, on1, od0, od1, hh0, hh1, sa, mrow, gat_b)


def bn2_f(z, st, bng, bnb, w2, dis):
    def blk(zr, str_, bngr, bnbr, wr, disr, o_f):
        mean = str_[0] / N
        var = str_[1] / N - mean * mean
        h = (zr[...] - mean) * lax.rsqrt(var + 1e-5) * bngr[...] + bnbr[...]
        h = jnp.maximum(h, 0.0)
        o_f[...] = jnp.dot(h, wr[...], preferred_element_type=jnp.float32) * disr[...]

    return pl.pallas_call(
        blk, grid=(GRID,),
        in_specs=[_rb(32), _full((2, 32)), _full((32,)), _full((32,)),
                  _full((32, 16)), _rb(1)],
        out_specs=_rb(16),
        out_shape=jax.ShapeDtypeStruct((N, 16), jnp.float32),
    )(z, st, bng, bnb, w2, dis)


def z3stat(q0, q1, f, dis, bias):
    def blk(q0r, q1r, fr, disr, br, o_z, o_st):
        z = disr[...] * (q0r[...] + q1r[...] + fr[...]) + br[...]
        o_z[...] = z

        @pl.when(pl.program_id(0) == 0)
        def _():
            o_st[...] = jnp.zeros_like(o_st)

        o_st[...] += jnp.stack([jnp.sum(z, axis=0), jnp.sum(z * z, axis=0)])

    return pl.pallas_call(
        blk, grid=(GRID,),
        in_specs=[_rb(16), _rb(16), _rb(16), _rb(1), _full((16,))],
        out_specs=[_rb(16), _full((2, 16))],
        out_shape=[jax.ShapeDtypeStruct((N, 16), jnp.float32),
                   jax.ShapeDtypeStruct((2, 16), jnp.float32)],
    )(q0, q1, f, dis, bias)


def final_head(z, st, bng, bnb, w1, b1, w2, b2):
    def blk(zr, str_, bngr, bnbr, w1r, b1r, w2r, b2r, o):
        mean = str_[0] / N
        var = str_[1] / N - mean * mean
        h = (zr[...] - mean) * lax.rsqrt(var + 1e-5) * bngr[...] + bnbr[...]
        h = jnp.maximum(h, 0.0)
        h = jnp.maximum(jnp.dot(h, w1r[...], preferred_element_type=jnp.float32) + b1r[...], 0.0)
        o[...] = jax.nn.sigmoid(jnp.dot(h, w2r[...], preferred_element_type=jnp.float32) + b2r[...])

    return pl.pallas_call(
        blk, grid=(GRID,),
        in_specs=[_rb(16), _full((2, 16)), _full((16,)), _full((16,)),
                  _full((16, 8)), _full((8,)), _full((8, 1)), _full((1,))],
        out_specs=_rb(1),
        out_shape=jax.ShapeDtypeStruct((N, 1), jnp.float32),
    )(z, st, bng, bnb, w1, b1, w2, b2)


def kernel(x, edge_index, ln_g, ln_b, enc_w1, enc_b1, enc_w2, enc_b2, gcn1_w, gcn1_b, bn1_g, bn1_b, gat_w, gat_as, gat_ad, gat_b, bn2_g, bn2_b, gcn2_w, gcn2_b, bn3_g, bn3_b, pr_w1, pr_b1, pr_w2, pr_b2):
    src = edge_index[0]
    dst = edge_index[1]
    src2d = jnp.concatenate(
        [src, jnp.zeros((EPAD,), jnp.int32)]).reshape(EROWS, 128)
    dst2d = jnp.concatenate(
        [dst, jnp.full((EPAD,), N, jnp.int32)]).reshape(EROWS, 128)

    h0 = enc(x, ln_g, ln_b, enc_w1, enc_b1, enc_w2, enc_b2)

    p0, p1 = _deg_call(dst2d)
    dis, g0, g1 = gprep(p0.reshape(N, 1), p1.reshape(N, 1), h0, gcn1_w)

    s0, s1 = _segsum_chsplit(src2d, dst2d, g0, g1)
    z1, st1 = zstat16x2(s0, s1, g0, g1, dis, gcn1_b)
    hh0, hh1, sa, mx = bn1_gat_prep(z1, st1, bn1_g, bn1_b, gat_w, gat_as, gat_ad)
    bound = mx[0, 0:2] + mx[0, 2:4]
    M = jnp.where(bound > 0, bound, 0.2 * bound)

    zpad8 = jnp.zeros((8,), jnp.float32)
    on0, on1, wout = _gat_call(
        src2d, dst2d, hh0, hh1,
        jnp.concatenate([sa[:, 0], zpad8]), jnp.concatenate([sa[:, 1], zpad8]),
        jnp.concatenate([sa[:, 2], zpad8]), jnp.concatenate([sa[:, 3], zpad8]),
        jnp.broadcast_to(M[:, None], (2, 16)))
    d0, d1 = _wden_call(wout, dst2d)
    z2, st2 = gat_epilogue(on0, on1, d0.reshape(N, 1), d1.reshape(N, 1),
                           hh0, hh1, sa, M.reshape(1, 2), gat_b)
    f = bn2_f(z2, st2, bn2_g, bn2_b, gcn2_w, dis)

    q0, q1 = _segsum_esplit(src2d, dst2d, f, f)
    z3, st3 = z3stat(q0, q1, f, dis, gcn2_b)
    out = final_head(z3, st3, bn3_g, bn3_b, pr_w1, pr_b1, pr_w2, pr_b2)
    return out.reshape(N)


# speedup vs baseline: 62.6761x; 1.0347x over previous
import jax
import jax.numpy as jnp
from jax import lax
from jax.experimental import pallas as pl
from jax.experimental.pallas import tpu as pltpu
from jax.experimental.pallas import tpu_sc as plsc

N = 100000
E = 1600000
HID = 32
HEADS = 2

NC = 2
NS = 16
LANES = 16
NW = NC * NS

WROWS = 392
EROWS = NW * WROWS
EPAD = EROWS * 128 - E

ZROWS = 6256
LASTZ = N - (NS - 1) * ZROWS

_MESH = plsc.VectorSubcoreMesh(core_axis_name="c", subcore_axis_name="s")


def _fill(ref, start, nvec, value):
    def body(i, _):
        ref[pl.ds(start + i * LANES, LANES)] = jnp.full((LANES,), value, ref.dtype)
        return 0

    lax.fori_loop(0, nvec, body, 0)


def _deg_body(dst2d, p0, p1, acc, idxbig, ones_row, zbuf, sem):
    cid = lax.axis_index("c")
    sid = lax.axis_index("s")
    wid = sid * NC + cid

    _fill(zbuf, 0, ZROWS // LANES, 0.0)
    _fill(ones_row, 0, 128 // LANES, 1.0)
    base = sid * ZROWS

    @pl.when(sid < NS - 1)
    def _():
        pltpu.sync_copy(zbuf.at[pl.ds(0, ZROWS)], acc.at[pl.ds(base, ZROWS)])

    @pl.when(sid == NS - 1)
    def _():
        pltpu.sync_copy(zbuf.at[pl.ds(0, LASTZ)], acc.at[pl.ds(base, LASTZ)])

    plsc.subcore_barrier()

    row0 = wid * WROWS
    pltpu.sync_copy(dst2d.at[pl.ds(row0, WROWS)], idxbig)

    def srow(j, _):
        pltpu.sync_copy(ones_row, acc.at[idxbig.at[j]], add=True)
        return 0

    lax.fori_loop(0, WROWS, srow, 0)
    plsc.subcore_barrier()

    @pl.when(sid < NS - 1)
    def _():
        pltpu.sync_copy(acc.at[pl.ds(base, ZROWS)], zbuf.at[pl.ds(0, ZROWS)])

    @pl.when(sid == NS - 1)
    def _():
        pltpu.sync_copy(acc.at[pl.ds(base, LASTZ)], zbuf.at[pl.ds(0, LASTZ)])

    @pl.when(cid == 0)
    def _():
        @pl.when(sid < NS - 1)
        def _():
            pltpu.sync_copy(zbuf.at[pl.ds(0, ZROWS)], p0.at[pl.ds(base, ZROWS)])

        @pl.when(sid == NS - 1)
        def _():
            pltpu.sync_copy(zbuf.at[pl.ds(0, LASTZ)], p0.at[pl.ds(base, LASTZ)])

    @pl.when(cid == 1)
    def _():
        @pl.when(sid < NS - 1)
        def _():
            pltpu.sync_copy(zbuf.at[pl.ds(0, ZROWS)], p1.at[pl.ds(base, ZROWS)])

        @pl.when(sid == NS - 1)
        def _():
            pltpu.sync_copy(zbuf.at[pl.ds(0, LASTZ)], p1.at[pl.ds(base, LASTZ)])


_deg_call = pl.kernel(
    _deg_body,
    out_type=[jax.ShapeDtypeStruct((N,), jnp.float32) for _ in range(2)],
    mesh=_MESH,
    scratch_types=[
        pltpu.VMEM_SHARED((N + 8,), jnp.float32),
        pltpu.VMEM((WROWS, 128), jnp.int32),
        pltpu.VMEM((128,), jnp.float32),
        pltpu.VMEM((ZROWS,), jnp.float32),
        pltpu.SemaphoreType.DMA,
    ],
)


_FCH = 784


def _row_chunks(total):
    full, rem = divmod(total, _FCH)
    out = [(i * _FCH, _FCH) for i in range(full)]
    if rem:
        out.append((full * _FCH, rem))
    return out


def _zero16(acc, zbuf, sid):
    def zrow(i, _):
        zbuf[i] = jnp.zeros((LANES,), jnp.float32)
        return 0

    lax.fori_loop(0, _FCH, zrow, 0)

    def do(total):
        for off, sz in _row_chunks(total):
            pltpu.sync_copy(zbuf.at[pl.ds(0, sz)],
                            acc.at[pl.ds(sid * ZROWS + off, sz)])

    @pl.when(sid < NS - 1)
    def _():
        do(ZROWS)

    @pl.when(sid == NS - 1)
    def _():
        do(LASTZ)


def _flush16(acc, out, zbuf, sid):
    def do(total):
        for off, sz in _row_chunks(total):
            pltpu.sync_copy(acc.at[pl.ds(sid * ZROWS + off, sz)],
                            zbuf.at[pl.ds(0, sz)])
            pltpu.sync_copy(zbuf.at[pl.ds(0, sz)],
                            out.at[pl.ds(sid * ZROWS + off, sz)])

    @pl.when(sid < NS - 1)
    def _():
        do(ZROWS)

    @pl.when(sid == NS - 1)
    def _():
        do(LASTZ)


def _zero1(acc, zbuf, sid):
    _fill(zbuf, 0, ZROWS // LANES, 0.0)
    base = sid * ZROWS

    @pl.when(sid < NS - 1)
    def _():
        pltpu.sync_copy(zbuf.at[pl.ds(0, ZROWS)], acc.at[pl.ds(base, ZROWS)])

    @pl.when(sid == NS - 1)
    def _():
        pltpu.sync_copy(zbuf.at[pl.ds(0, LASTZ)], acc.at[pl.ds(base, LASTZ)])


def _flush1(acc, out, zbuf, sid):
    base = sid * ZROWS

    @pl.when(sid < NS - 1)
    def _():
        pltpu.sync_copy(acc.at[pl.ds(base, ZROWS)], zbuf.at[pl.ds(0, ZROWS)])
        pltpu.sync_copy(zbuf.at[pl.ds(0, ZROWS)], out.at[pl.ds(base, ZROWS)])

    @pl.when(sid == NS - 1)
    def _():
        pltpu.sync_copy(acc.at[pl.ds(base, LASTZ)], zbuf.at[pl.ds(0, LASTZ)])
        pltpu.sync_copy(zbuf.at[pl.ds(0, LASTZ)], out.at[pl.ds(base, LASTZ)])


_SCH = 8


def _make_segsum(rows_per_tile, edge_split):
    nchunks = rows_per_tile // _SCH

    def body(src2d, dst2d, g0, g1, o0, o1, acc, sidx, didx, rows, zbuf, sem, sem2):
        cid = lax.axis_index("c")
        sid = lax.axis_index("s")
        _zero16(acc, zbuf, sid)
        plsc.subcore_barrier()

        if edge_split:
            row0 = cid * (EROWS // 2) + sid * rows_per_tile
        else:
            row0 = sid * rows_per_tile

        def inner(gtab):
            def chunk(ci, _):
                r = row0 + ci * _SCH
                pltpu.sync_copy(src2d.at[pl.ds(r, _SCH)], sidx)
                pltpu.sync_copy(dst2d.at[pl.ds(r, _SCH)], didx)
                descs = [pltpu.async_copy(gtab.at[sidx.at[j]], rows.at[j], sem)
                         for j in range(_SCH)]
                for d in descs:
                    d.wait()
                for j in range(_SCH):
                    pltpu.sync_copy(rows.at[j], acc.at[didx.at[j]], add=True)
                return 0

            lax.fori_loop(0, nchunks, chunk, 0)

        @pl.when(cid == 0)
        def _():
            inner(g0)

        @pl.when(cid == 1)
        def _():
            inner(g1)

        plsc.subcore_barrier()

        @pl.when(cid == 0)
        def _():
            _flush16(acc, o0, zbuf, sid)

        @pl.when(cid == 1)
        def _():
            _flush16(acc, o1, zbuf, sid)

    return pl.kernel(
        body,
        out_type=[jax.ShapeDtypeStruct((N, 16), jnp.float32) for _ in range(2)],
        mesh=_MESH,
        scratch_types=[
            pltpu.VMEM_SHARED((N + 8, 16), jnp.float32),
            pltpu.VMEM((_SCH, 128), jnp.int32),
            pltpu.VMEM((_SCH, 128), jnp.int32),
            pltpu.VMEM((_SCH, 128, 16), jnp.float32),
            pltpu.VMEM((_FCH, 16), jnp.float32),
            pltpu.SemaphoreType.DMA,
            pltpu.SemaphoreType.DMA,
        ],
        compiler_params=pltpu.CompilerParams(use_tc_tiling_on_sc=False, needs_layout_passes=False),
    )


_segsum_chsplit = _make_segsum(EROWS // NS, False)
_segsum_esplit = _make_segsum(EROWS // NW, True)


_GCH = 4


def _gat_body(src2d, dst2d, hh0, hh1, as0, as1, ad0, ad1, mrows,
              on0, on1, wout,
              accn, sidx, didx, asb, adb, hhb, wc, mvb, zbuf, sem, sem2):
    cid = lax.axis_index("c")
    sid = lax.axis_index("s")
    _zero16(accn, zbuf, sid)
    pltpu.sync_copy(mrows.at[pl.ds(cid, 1)], mvb)
    plsc.subcore_barrier()

    iota = lax.iota(jnp.int32, LANES)
    mv = mvb[0]
    row0 = sid * (EROWS // NS)
    nchunks = (EROWS // NS) // _GCH

    def inner(hhtab, astab, adtab, wbase):
        def chunk(ci, _):
            r = row0 + ci * _GCH
            pltpu.sync_copy(src2d.at[pl.ds(r, _GCH)], sidx)
            pltpu.sync_copy(dst2d.at[pl.ds(r, _GCH)], didx)
            descs = []
            for j in range(_GCH):
                descs.append((pltpu.async_copy(astab.at[sidx.at[j]], asb.at[j], sem),
                              pltpu.async_copy(adtab.at[didx.at[j]], adb.at[j], sem),
                              pltpu.async_copy(hhtab.at[sidx.at[j]], hhb.at[j], sem)))

            for j in range(_GCH):
                for d in descs[j]:
                    d.wait()

                def grp(gidx, _):
                    e0 = gidx * LANES
                    ev = asb[j, pl.ds(e0, LANES)] + adb[j, pl.ds(e0, LANES)]
                    ev = jnp.where(ev > 0.0, ev, 0.2 * ev) - mv
                    wv = jnp.exp(ev)
                    wc[j, pl.ds(e0, LANES)] = wv
                    idx_e = iota + e0
                    for k in range(16):
                        idx_k = jnp.full((LANES,), k, jnp.int32)
                        hv = plsc.load_gather(hhb.at[j], [idx_e, idx_k])
                        plsc.store_scatter(hhb.at[j], [idx_e, idx_k], hv * wv)
                    return 0

                lax.fori_loop(0, 128 // LANES, grp, 0)
                if j > 0:
                    pltpu.make_async_copy(hhb.at[j - 1], accn.at[didx.at[j - 1]],
                                          sem2).wait()
                pltpu.async_copy(hhb.at[j], accn.at[didx.at[j]], sem2, add=True)

            pltpu.sync_copy(wc, wout.at[pl.ds(wbase + r, _GCH)])
            pltpu.make_async_copy(hhb.at[_GCH - 1], accn.at[didx.at[_GCH - 1]],
                                  sem2).wait()
            return 0

        lax.fori_loop(0, nchunks, chunk, 0)

    @pl.when(cid == 0)
    def _():
        inner(hh0, as0, ad0, 0)

    @pl.when(cid == 1)
    def _():
        inner(hh1, as1, ad1, EROWS)

    plsc.subcore_barrier()

    @pl.when(cid == 0)
    def _():
        _flush16(accn, on0, zbuf, sid)

    @pl.when(cid == 1)
    def _():
        _flush16(accn, on1, zbuf, sid)


_gat_call = pl.kernel(
    _gat_body,
    out_type=[jax.ShapeDtypeStruct((N, 16), jnp.float32),
              jax.ShapeDtypeStruct((N, 16), jnp.float32),
              jax.ShapeDtypeStruct((2 * EROWS, 128), jnp.float32)],
    mesh=_MESH,
    scratch_types=[
        pltpu.VMEM_SHARED((N + 8, 16), jnp.float32),
        pltpu.VMEM((_GCH, 128), jnp.int32),
        pltpu.VMEM((_GCH, 128), jnp.int32),
        pltpu.VMEM((_GCH, 128), jnp.float32),
        pltpu.VMEM((_GCH, 128), jnp.float32),
        pltpu.VMEM((_GCH, 128, 16), jnp.float32),
        pltpu.VMEM((_GCH, 128), jnp.float32),
        pltpu.VMEM((1, 16), jnp.float32),
        pltpu.VMEM((_FCH, 16), jnp.float32),
        pltpu.SemaphoreType.DMA,
        pltpu.SemaphoreType.DMA,
    ],
    compiler_params=pltpu.CompilerParams(use_tc_tiling_on_sc=False, needs_layout_passes=False),
)


_DCH = 8


def _wden_body(w2d, dst2d, d0, d1, acc, widx, wval, zbuf, sem, sem2):
    cid = lax.axis_index("c")
    sid = lax.axis_index("s")
    _zero1(acc, zbuf, sid)
    plsc.subcore_barrier()

    row0 = sid * (EROWS // NS)
    nchunks = (EROWS // NS) // _DCH
    wbase = cid * EROWS

    def chunk(ci, _):
        r = row0 + ci * _DCH
        pltpu.sync_copy(dst2d.at[pl.ds(r, _DCH)], widx)
        pltpu.sync_copy(w2d.at[pl.ds(wbase + r, _DCH)], wval)
        for j in range(_DCH):
            pltpu.sync_copy(wval.at[j], acc.at[widx.at[j]], add=True)
        return 0

    lax.fori_loop(0, nchunks, chunk, 0)
    plsc.subcore_barrier()

    @pl.when(cid == 0)
    def _():
        _flush1(acc, d0, zbuf, sid)

    @pl.when(cid == 1)
    def _():
        _flush1(acc, d1, zbuf, sid)


_wden_call = pl.kernel(
    _wden_body,
    out_type=[jax.ShapeDtypeStruct((N,), jnp.float32) for _ in range(2)],
    mesh=_MESH,
    scratch_types=[
        pltpu.VMEM_SHARED((N + 8,), jnp.float32),
        pltpu.VMEM((_DCH, 128), jnp.int32),
        pltpu.VMEM((_DCH, 128), jnp.float32),
        pltpu.VMEM((ZROWS,), jnp.float32),
        pltpu.SemaphoreType.DMA,
        pltpu.SemaphoreType.DMA,
    ],
    compiler_params=pltpu.CompilerParams(use_tc_tiling_on_sc=False, needs_layout_passes=False),
)


B = 2000
GRID = N // B


def _rb(width):
    return pl.BlockSpec((B, width), lambda i: (i, 0))


def _full(shape):
    return pl.BlockSpec(shape, lambda i: tuple(0 for _ in shape))


def _enc_block(x_ref, lng, lnb, w1, b1, w2, b2, o_ref):
    x = x_ref[...]
    m = jnp.mean(x, axis=1, keepdims=True)
    v = jnp.mean((x - m) ** 2, axis=1, keepdims=True)
    h = (x - m) * lax.rsqrt(v + 1e-5) * lng[...] + lnb[...]
    h = jnp.maximum(jnp.dot(h, w1[...], preferred_element_type=jnp.float32) + b1[...], 0.0)
    h = jnp.maximum(jnp.dot(h, w2[...], preferred_element_type=jnp.float32) + b2[...], 0.0)
    o_ref[...] = h


def enc(x, lng, lnb, w1, b1, w2, b2):
    return pl.pallas_call(
        _enc_block, grid=(GRID,),
        in_specs=[_rb(6), _full((6,)), _full((6,)), _full((6, 32)),
                  _full((32,)), _full((32, 32)), _full((32,))],
        out_specs=_rb(32),
        out_shape=jax.ShapeDtypeStruct((N, 32), jnp.float32),
    )(x, lng, lnb, w1, b1, w2, b2)


def _g_block(p0, p1, h0, w, o_dis, o_g0, o_g1):
    deg = p0[...] + p1[...] + 1.0
    dis = lax.rsqrt(deg)
    g = jnp.dot(h0[...], w[...], preferred_element_type=jnp.float32) * dis
    o_dis[...] = dis
    o_g0[...] = g[:, :16]
    o_g1[...] = g[:, 16:]


def gprep(p0, p1, h0, w):
    return pl.pallas_call(
        _g_block, grid=(GRID,),
        in_specs=[_rb(1), _rb(1), _rb(32), _full((32, 32))],
        out_specs=[_rb(1), _rb(16), _rb(16)],
        out_shape=[jax.ShapeDtypeStruct((N, 1), jnp.float32),
                   jax.ShapeDtypeStruct((N, 16), jnp.float32),
                   jax.ShapeDtypeStruct((N, 16), jnp.float32)],
    )(p0, p1, h0, w)


def zstat16x2(s0, s1, g0, g1, dis, bias):
    def blk(s0r, s1r, g0r, g1r, disr, br, o_z, o_st):
        d = disr[...]
        z = jnp.concatenate([d * (s0r[...] + g0r[...]),
                             d * (s1r[...] + g1r[...])], axis=1) + br[...]
        o_z[...] = z

        @pl.when(pl.program_id(0) == 0)
        def _():
            o_st[...] = jnp.zeros_like(o_st)

        o_st[...] += jnp.stack([jnp.sum(z, axis=0), jnp.sum(z * z, axis=0)])

    return pl.pallas_call(
        blk, grid=(GRID,),
        in_specs=[_rb(16), _rb(16), _rb(16), _rb(16), _rb(1), _full((32,))],
        out_specs=[_rb(32), _full((2, 32))],
        out_shape=[jax.ShapeDtypeStruct((N, 32), jnp.float32),
                   jax.ShapeDtypeStruct((2, 32), jnp.float32)],
    )(s0, s1, g0, g1, dis, bias)


def bn1_gat_prep(z, st, bng, bnb, gat_w, gat_as, gat_ad):
    def blk(zr, str_, bngr, bnbr, wr, asr, adr, o_h0, o_h1, o_sa, o_mx):
        mean = str_[0] / N
        var = str_[1] / N - mean * mean
        h = (zr[...] - mean) * lax.rsqrt(var + 1e-5) * bngr[...] + bnbr[...]
        h = jnp.maximum(h, 0.0)
        hh = jnp.dot(h, wr[...], preferred_element_type=jnp.float32)
        hh0 = hh[:, :16]
        hh1 = hh[:, 16:]
        a0 = jnp.sum(hh0 * asr[0], axis=1)
        a1 = jnp.sum(hh1 * asr[1], axis=1)
        d0 = jnp.sum(hh0 * adr[0], axis=1)
        d1 = jnp.sum(hh1 * adr[1], axis=1)
        o_h0[...] = hh0
        o_h1[...] = hh1
        o_sa[...] = jnp.stack([a0, a1, d0, d1], axis=1)

        @pl.when(pl.program_id(0) == 0)
        def _():
            o_mx[...] = jnp.full_like(o_mx, -3.0e38)

        mx = jnp.stack([jnp.max(a0), jnp.max(a1), jnp.max(d0), jnp.max(d1)])
        o_mx[...] = jnp.maximum(o_mx[...], mx[None, :])

    return pl.pallas_call(
        blk, grid=(GRID,),
        in_specs=[_rb(32), _full((2, 32)), _full((32,)), _full((32,)),
                  _full((32, 32)), _full((2, 16)), _full((2, 16))],
        out_specs=[_rb(16), _rb(16), _rb(4), _full((1, 4))],
        out_shape=[jax.ShapeDtypeStruct((N, 16), jnp.float32),
                   jax.ShapeDtypeStruct((N, 16), jnp.float32),
                   jax.ShapeDtypeStruct((N, 4), jnp.float32),
                   jax.ShapeDtypeStruct((1, 4), jnp.float32)],
    )(z, st, bng, bnb, gat_w, gat_as, gat_ad)


def gat_epilogue(on0, on1, od0, od1, hh0, hh1, sa, mrow, gat_b):
    def blk(on0r, on1r, od0r, od1r, h0r, h1r, sar, mr, br, o_z, o_st):
        es = sar[..., 0:2] + sar[..., 2:4]
        ws = jnp.exp(jnp.where(es > 0, es, 0.2 * es) - mr[...])
        n0 = on0r[...] + h0r[...] * ws[:, 0:1]
        n1 = on1r[...] + h1r[...] * ws[:, 1:2]
        d0 = od0r[...] + ws[:, 0:1]
        d1 = od1r[...] + ws[:, 1:2]
        z = jnp.concatenate([n0 / (d0 + 1e-16), n1 / (d1 + 1e-16)], axis=1) + br[...]
        o_z[...] = z

        @pl.when(pl.program_id(0) == 0)
        def _():
            o_st[...] = jnp.zeros_like(o_st)

        o_st[...] += jnp.stack([jnp.sum(z, axis=0), jnp.sum(z * z, axis=0)])

    return pl.pallas_call(
        blk, grid=(GRID,),
        in_specs=[_rb(16), _rb(16), _rb(1), _rb(1), _rb(16), _rb(16),
                  _rb(4), _full((1, 2)), _full((32,))],
        out_specs=[_rb(32), _full((2, 32))],
        out_shape=[jax.ShapeDtypeStruct((N, 32), jnp.float32),
                   jax.ShapeDtypeStruct((2, 32), jnp.float32)],
    )(on0, on1, od0, od1, hh0, hh1, sa, mrow, gat_b)


def bn2_f(z, st, bng, bnb, w2, dis):
    def blk(zr, str_, bngr, bnbr, wr, disr, o_f):
        mean = str_[0] / N
        var = str_[1] / N - mean * mean
        h = (zr[...] - mean) * lax.rsqrt(var + 1e-5) * bngr[...] + bnbr[...]
        h = jnp.maximum(h, 0.0)
        o_f[...] = jnp.dot(h, wr[...], preferred_element_type=jnp.float32) * disr[...]

    return pl.pallas_call(
        blk, grid=(GRID,),
        in_specs=[_rb(32), _full((2, 32)), _full((32,)), _full((32,)),
                  _full((32, 16)), _rb(1)],
        out_specs=_rb(16),
        out_shape=jax.ShapeDtypeStruct((N, 16), jnp.float32),
    )(z, st, bng, bnb, w2, dis)


def z3stat(q0, q1, f, dis, bias):
    def blk(q0r, q1r, fr, disr, br, o_z, o_st):
        z = disr[...] * (q0r[...] + q1r[...] + fr[...]) + br[...]
        o_z[...] = z

        @pl.when(pl.program_id(0) == 0)
        def _():
            o_st[...] = jnp.zeros_like(o_st)

        o_st[...] += jnp.stack([jnp.sum(z, axis=0), jnp.sum(z * z, axis=0)])

    return pl.pallas_call(
        blk, grid=(GRID,),
        in_specs=[_rb(16), _rb(16), _rb(16), _rb(1), _full((16,))],
        out_specs=[_rb(16), _full((2, 16))],
        out_shape=[jax.ShapeDtypeStruct((N, 16), jnp.float32),
                   jax.ShapeDtypeStruct((2, 16), jnp.float32)],
    )(q0, q1, f, dis, bias)


def final_head(z, st, bng, bnb, w1, b1, w2, b2):
    def blk(zr, str_, bngr, bnbr, w1r, b1r, w2r, b2r, o):
        mean = str_[0] / N
        var = str_[1] / N - mean * mean
        h = (zr[...] - mean) * lax.rsqrt(var + 1e-5) * bngr[...] + bnbr[...]
        h = jnp.maximum(h, 0.0)
        h = jnp.maximum(jnp.dot(h, w1r[...], preferred_element_type=jnp.float32) + b1r[...], 0.0)
        o[...] = jax.nn.sigmoid(jnp.dot(h, w2r[...], preferred_element_type=jnp.float32) + b2r[...])

    return pl.pallas_call(
        blk, grid=(GRID,),
        in_specs=[_rb(16), _full((2, 16)), _full((16,)), _full((16,)),
                  _full((16, 8)), _full((8,)), _full((8, 1)), _full((1,))],
        out_specs=_rb(1),
        out_shape=jax.ShapeDtypeStruct((N, 1), jnp.float32),
    )(z, st, bng, bnb, w1, b1, w2, b2)


def kernel(x, edge_index, ln_g, ln_b, enc_w1, enc_b1, enc_w2, enc_b2, gcn1_w, gcn1_b, bn1_g, bn1_b, gat_w, gat_as, gat_ad, gat_b, bn2_g, bn2_b, gcn2_w, gcn2_b, bn3_g, bn3_b, pr_w1, pr_b1, pr_w2, pr_b2):
    src = edge_index[0]
    dst = edge_index[1]
    src2d = jnp.concatenate(
        [src, jnp.zeros((EPAD,), jnp.int32)]).reshape(EROWS, 128)
    dst2d = jnp.concatenate(
        [dst, jnp.full((EPAD,), N, jnp.int32)]).reshape(EROWS, 128)

    h0 = enc(x, ln_g, ln_b, enc_w1, enc_b1, enc_w2, enc_b2)

    p0, p1 = _deg_call(dst2d)
    dis, g0, g1 = gprep(p0.reshape(N, 1), p1.reshape(N, 1), h0, gcn1_w)

    s0, s1 = _segsum_chsplit(src2d, dst2d, g0, g1)
    z1, st1 = zstat16x2(s0, s1, g0, g1, dis, gcn1_b)
    hh0, hh1, sa, mx = bn1_gat_prep(z1, st1, bn1_g, bn1_b, gat_w, gat_as, gat_ad)
    bound = mx[0, 0:2] + mx[0, 2:4]
    M = jnp.where(bound > 0, bound, 0.2 * bound)

    zpad8 = jnp.zeros((8,), jnp.float32)
    on0, on1, wout = _gat_call(
        src2d, dst2d, hh0, hh1,
        jnp.concatenate([sa[:, 0], zpad8]), jnp.concatenate([sa[:, 1], zpad8]),
        jnp.concatenate([sa[:, 2], zpad8]), jnp.concatenate([sa[:, 3], zpad8]),
        jnp.broadcast_to(M[:, None], (2, 16)))
    d0, d1 = _wden_call(wout, dst2d)
    z2, st2 = gat_epilogue(on0, on1, d0.reshape(N, 1), d1.reshape(N, 1),
                           hh0, hh1, sa, M.reshape(1, 2), gat_b)
    f = bn2_f(z2, st2, bn2_g, bn2_b, gcn2_w, dis)

    q0, q1 = _segsum_esplit(src2d, dst2d, f, f)
    z3, st3 = z3stat(q0, q1, f, dis, gcn2_b)
    out = final_head(z3, st3, bn3_g, bn3_b, pr_w1, pr_b1, pr_w2, pr_b2)
    return out.reshape(N)


# speedup vs baseline: 63.8832x; 1.0193x over previous
import jax
import jax.numpy as jnp
from jax import lax
from jax.experimental import pallas as pl
from jax.experimental.pallas import tpu as pltpu
from jax.experimental.pallas import tpu_sc as plsc

N = 100000
E = 1600000
HID = 32
HEADS = 2

NC = 2
NS = 16
LANES = 16
NW = NC * NS

WROWS = 392
EROWS = NW * WROWS
EPAD = EROWS * 128 - E

ZROWS = 6256
LASTZ = N - (NS - 1) * ZROWS

_MESH = plsc.VectorSubcoreMesh(core_axis_name="c", subcore_axis_name="s")


def _fill(ref, start, nvec, value):
    def body(i, _):
        ref[pl.ds(start + i * LANES, LANES)] = jnp.full((LANES,), value, ref.dtype)
        return 0

    lax.fori_loop(0, nvec, body, 0)


def _deg_body(dst2d, p0, p1, acc, idxbig, ones_row, zbuf, sem):
    cid = lax.axis_index("c")
    sid = lax.axis_index("s")
    wid = sid * NC + cid

    _fill(zbuf, 0, ZROWS // LANES, 0.0)
    _fill(ones_row, 0, 128 // LANES, 1.0)
    base = sid * ZROWS

    @pl.when(sid < NS - 1)
    def _():
        pltpu.sync_copy(zbuf.at[pl.ds(0, ZROWS)], acc.at[pl.ds(base, ZROWS)])

    @pl.when(sid == NS - 1)
    def _():
        pltpu.sync_copy(zbuf.at[pl.ds(0, LASTZ)], acc.at[pl.ds(base, LASTZ)])

    plsc.subcore_barrier()

    row0 = wid * WROWS
    pltpu.sync_copy(dst2d.at[pl.ds(row0, WROWS)], idxbig)

    def srow(j, _):
        pltpu.sync_copy(ones_row, acc.at[idxbig.at[j]], add=True)
        return 0

    lax.fori_loop(0, WROWS, srow, 0)
    plsc.subcore_barrier()

    @pl.when(sid < NS - 1)
    def _():
        pltpu.sync_copy(acc.at[pl.ds(base, ZROWS)], zbuf.at[pl.ds(0, ZROWS)])

    @pl.when(sid == NS - 1)
    def _():
        pltpu.sync_copy(acc.at[pl.ds(base, LASTZ)], zbuf.at[pl.ds(0, LASTZ)])

    @pl.when(cid == 0)
    def _():
        @pl.when(sid < NS - 1)
        def _():
            pltpu.sync_copy(zbuf.at[pl.ds(0, ZROWS)], p0.at[pl.ds(base, ZROWS)])

        @pl.when(sid == NS - 1)
        def _():
            pltpu.sync_copy(zbuf.at[pl.ds(0, LASTZ)], p0.at[pl.ds(base, LASTZ)])

    @pl.when(cid == 1)
    def _():
        @pl.when(sid < NS - 1)
        def _():
            pltpu.sync_copy(zbuf.at[pl.ds(0, ZROWS)], p1.at[pl.ds(base, ZROWS)])

        @pl.when(sid == NS - 1)
        def _():
            pltpu.sync_copy(zbuf.at[pl.ds(0, LASTZ)], p1.at[pl.ds(base, LASTZ)])


_deg_call = pl.kernel(
    _deg_body,
    out_type=[jax.ShapeDtypeStruct((N,), jnp.float32) for _ in range(2)],
    mesh=_MESH,
    scratch_types=[
        pltpu.VMEM_SHARED((N + 8,), jnp.float32),
        pltpu.VMEM((WROWS, 128), jnp.int32),
        pltpu.VMEM((128,), jnp.float32),
        pltpu.VMEM((ZROWS,), jnp.float32),
        pltpu.SemaphoreType.DMA,
    ],
)


_FCH = 784


def _row_chunks(total):
    full, rem = divmod(total, _FCH)
    out = [(i * _FCH, _FCH) for i in range(full)]
    if rem:
        out.append((full * _FCH, rem))
    return out


def _zero16(acc, zbuf, sid):
    def zrow(i, _):
        zbuf[i] = jnp.zeros((LANES,), jnp.float32)
        return 0

    lax.fori_loop(0, _FCH, zrow, 0)

    def do(total):
        for off, sz in _row_chunks(total):
            pltpu.sync_copy(zbuf.at[pl.ds(0, sz)],
                            acc.at[pl.ds(sid * ZROWS + off, sz)])

    @pl.when(sid < NS - 1)
    def _():
        do(ZROWS)

    @pl.when(sid == NS - 1)
    def _():
        do(LASTZ)


def _flush16(acc, out, zbuf, sid):
    def do(total):
        for off, sz in _row_chunks(total):
            pltpu.sync_copy(acc.at[pl.ds(sid * ZROWS + off, sz)],
                            zbuf.at[pl.ds(0, sz)])
            pltpu.sync_copy(zbuf.at[pl.ds(0, sz)],
                            out.at[pl.ds(sid * ZROWS + off, sz)])

    @pl.when(sid < NS - 1)
    def _():
        do(ZROWS)

    @pl.when(sid == NS - 1)
    def _():
        do(LASTZ)


def _zero1(acc, zbuf, sid):
    _fill(zbuf, 0, ZROWS // LANES, 0.0)
    base = sid * ZROWS

    @pl.when(sid < NS - 1)
    def _():
        pltpu.sync_copy(zbuf.at[pl.ds(0, ZROWS)], acc.at[pl.ds(base, ZROWS)])

    @pl.when(sid == NS - 1)
    def _():
        pltpu.sync_copy(zbuf.at[pl.ds(0, LASTZ)], acc.at[pl.ds(base, LASTZ)])


def _flush1(acc, out, zbuf, sid):
    base = sid * ZROWS

    @pl.when(sid < NS - 1)
    def _():
        pltpu.sync_copy(acc.at[pl.ds(base, ZROWS)], zbuf.at[pl.ds(0, ZROWS)])
        pltpu.sync_copy(zbuf.at[pl.ds(0, ZROWS)], out.at[pl.ds(base, ZROWS)])

    @pl.when(sid == NS - 1)
    def _():
        pltpu.sync_copy(acc.at[pl.ds(base, LASTZ)], zbuf.at[pl.ds(0, LASTZ)])
        pltpu.sync_copy(zbuf.at[pl.ds(0, LASTZ)], out.at[pl.ds(base, LASTZ)])


_SCH = 8


def _make_segsum(rows_per_tile, edge_split):
    nchunks = rows_per_tile // _SCH

    def body(src2d, dst2d, g0, g1, o0, o1, acc, sidx, didx, rows, zbuf, sem, sem2):
        cid = lax.axis_index("c")
        sid = lax.axis_index("s")
        _zero16(acc, zbuf, sid)
        plsc.subcore_barrier()

        if edge_split:
            row0 = cid * (EROWS // 2) + sid * rows_per_tile
        else:
            row0 = sid * rows_per_tile

        def inner(gtab):
            def chunk(ci, _):
                r = row0 + ci * _SCH
                pltpu.sync_copy(src2d.at[pl.ds(r, _SCH)], sidx)
                pltpu.sync_copy(dst2d.at[pl.ds(r, _SCH)], didx)
                descs = [pltpu.async_copy(gtab.at[sidx.at[j]], rows.at[j], sem)
                         for j in range(_SCH)]
                for j in range(_SCH):
                    descs[j].wait()
                    if j > 0:
                        pltpu.make_async_copy(rows.at[j - 1],
                                              acc.at[didx.at[j - 1]], sem2).wait()
                    pltpu.async_copy(rows.at[j], acc.at[didx.at[j]], sem2, add=True)
                pltpu.make_async_copy(rows.at[_SCH - 1],
                                      acc.at[didx.at[_SCH - 1]], sem2).wait()
                return 0

            lax.fori_loop(0, nchunks, chunk, 0)

        @pl.when(cid == 0)
        def _():
            inner(g0)

        @pl.when(cid == 1)
        def _():
            inner(g1)

        plsc.subcore_barrier()

        @pl.when(cid == 0)
        def _():
            _flush16(acc, o0, zbuf, sid)

        @pl.when(cid == 1)
        def _():
            _flush16(acc, o1, zbuf, sid)

    return pl.kernel(
        body,
        out_type=[jax.ShapeDtypeStruct((N, 16), jnp.float32) for _ in range(2)],
        mesh=_MESH,
        scratch_types=[
            pltpu.VMEM_SHARED((N + 8, 16), jnp.float32),
            pltpu.VMEM((_SCH, 128), jnp.int32),
            pltpu.VMEM((_SCH, 128), jnp.int32),
            pltpu.VMEM((_SCH, 128, 16), jnp.float32),
            pltpu.VMEM((_FCH, 16), jnp.float32),
            pltpu.SemaphoreType.DMA,
            pltpu.SemaphoreType.DMA,
        ],
        compiler_params=pltpu.CompilerParams(use_tc_tiling_on_sc=False, needs_layout_passes=False),
    )


_segsum_chsplit = _make_segsum(EROWS // NS, False)
_segsum_esplit = _make_segsum(EROWS // NW, True)


_GCH = 4


def _gat_body(src2d, dst2d, hh0, hh1, as0, as1, ad0, ad1, mrows,
              on0, on1, wout,
              accn, sidx, didx, asb, adb, hhb, wc, mvb, zbuf, sem, sem2):
    cid = lax.axis_index("c")
    sid = lax.axis_index("s")
    _zero16(accn, zbuf, sid)
    pltpu.sync_copy(mrows.at[pl.ds(cid, 1)], mvb)
    plsc.subcore_barrier()

    iota = lax.iota(jnp.int32, LANES)
    mv = mvb[0]
    row0 = sid * (EROWS // NS)
    nchunks = (EROWS // NS) // _GCH

    def inner(hhtab, astab, adtab, wbase):
        def chunk(ci, _):
            r = row0 + ci * _GCH
            pltpu.sync_copy(src2d.at[pl.ds(r, _GCH)], sidx)
            pltpu.sync_copy(dst2d.at[pl.ds(r, _GCH)], didx)
            descs = []
            for j in range(_GCH):
                descs.append((pltpu.async_copy(astab.at[sidx.at[j]], asb.at[j], sem),
                              pltpu.async_copy(adtab.at[didx.at[j]], adb.at[j], sem),
                              pltpu.async_copy(hhtab.at[sidx.at[j]], hhb.at[j], sem)))

            for j in range(_GCH):
                for d in descs[j]:
                    d.wait()

                def grp(gidx, _):
                    e0 = gidx * LANES
                    ev = asb[j, pl.ds(e0, LANES)] + adb[j, pl.ds(e0, LANES)]
                    ev = jnp.where(ev > 0.0, ev, 0.2 * ev) - mv
                    wv = jnp.exp(ev)
                    wc[j, pl.ds(e0, LANES)] = wv
                    idx_e = iota + e0
                    for k in range(16):
                        idx_k = jnp.full((LANES,), k, jnp.int32)
                        hv = plsc.load_gather(hhb.at[j], [idx_e, idx_k])
                        plsc.store_scatter(hhb.at[j], [idx_e, idx_k], hv * wv)
                    return 0

                lax.fori_loop(0, 128 // LANES, grp, 0)
                if j > 0:
                    pltpu.make_async_copy(hhb.at[j - 1], accn.at[didx.at[j - 1]],
                                          sem2).wait()
                pltpu.async_copy(hhb.at[j], accn.at[didx.at[j]], sem2, add=True)

            pltpu.sync_copy(wc, wout.at[pl.ds(wbase + r, _GCH)])
            pltpu.make_async_copy(hhb.at[_GCH - 1], accn.at[didx.at[_GCH - 1]],
                                  sem2).wait()
            return 0

        lax.fori_loop(0, nchunks, chunk, 0)

    @pl.when(cid == 0)
    def _():
        inner(hh0, as0, ad0, 0)

    @pl.when(cid == 1)
    def _():
        inner(hh1, as1, ad1, EROWS)

    plsc.subcore_barrier()

    @pl.when(cid == 0)
    def _():
        _flush16(accn, on0, zbuf, sid)

    @pl.when(cid == 1)
    def _():
        _flush16(accn, on1, zbuf, sid)


_gat_call = pl.kernel(
    _gat_body,
    out_type=[jax.ShapeDtypeStruct((N, 16), jnp.float32),
              jax.ShapeDtypeStruct((N, 16), jnp.float32),
              jax.ShapeDtypeStruct((2 * EROWS, 128), jnp.float32)],
    mesh=_MESH,
    scratch_types=[
        pltpu.VMEM_SHARED((N + 8, 16), jnp.float32),
        pltpu.VMEM((_GCH, 128), jnp.int32),
        pltpu.VMEM((_GCH, 128), jnp.int32),
        pltpu.VMEM((_GCH, 128), jnp.float32),
        pltpu.VMEM((_GCH, 128), jnp.float32),
        pltpu.VMEM((_GCH, 128, 16), jnp.float32),
        pltpu.VMEM((_GCH, 128), jnp.float32),
        pltpu.VMEM((1, 16), jnp.float32),
        pltpu.VMEM((_FCH, 16), jnp.float32),
        pltpu.SemaphoreType.DMA,
        pltpu.SemaphoreType.DMA,
    ],
    compiler_params=pltpu.CompilerParams(use_tc_tiling_on_sc=False, needs_layout_passes=False),
)


_DCH = 8


def _wden_body(w2d, dst2d, d0, d1, acc, widx, wval, zbuf, sem, sem2):
    cid = lax.axis_index("c")
    sid = lax.axis_index("s")
    _zero1(acc, zbuf, sid)
    plsc.subcore_barrier()

    row0 = sid * (EROWS // NS)
    nchunks = (EROWS // NS) // _DCH
    wbase = cid * EROWS

    def chunk(ci, _):
        r = row0 + ci * _DCH
        pltpu.sync_copy(dst2d.at[pl.ds(r, _DCH)], widx)
        pltpu.sync_copy(w2d.at[pl.ds(wbase + r, _DCH)], wval)
        for j in range(_DCH):
            if j > 0:
                pltpu.make_async_copy(wval.at[j - 1], acc.at[widx.at[j - 1]],
                                      sem2).wait()
            pltpu.async_copy(wval.at[j], acc.at[widx.at[j]], sem2, add=True)
        pltpu.make_async_copy(wval.at[_DCH - 1], acc.at[widx.at[_DCH - 1]],
                              sem2).wait()
        return 0

    lax.fori_loop(0, nchunks, chunk, 0)
    plsc.subcore_barrier()

    @pl.when(cid == 0)
    def _():
        _flush1(acc, d0, zbuf, sid)

    @pl.when(cid == 1)
    def _():
        _flush1(acc, d1, zbuf, sid)


_wden_call = pl.kernel(
    _wden_body,
    out_type=[jax.ShapeDtypeStruct((N,), jnp.float32) for _ in range(2)],
    mesh=_MESH,
    scratch_types=[
        pltpu.VMEM_SHARED((N + 8,), jnp.float32),
        pltpu.VMEM((_DCH, 128), jnp.int32),
        pltpu.VMEM((_DCH, 128), jnp.float32),
        pltpu.VMEM((ZROWS,), jnp.float32),
        pltpu.SemaphoreType.DMA,
        pltpu.SemaphoreType.DMA,
    ],
    compiler_params=pltpu.CompilerParams(use_tc_tiling_on_sc=False, needs_layout_passes=False),
)


B = 2000
GRID = N // B


def _rb(width):
    return pl.BlockSpec((B, width), lambda i: (i, 0))


def _full(shape):
    return pl.BlockSpec(shape, lambda i: tuple(0 for _ in shape))


def _enc_block(x_ref, lng, lnb, w1, b1, w2, b2, o_ref):
    x = x_ref[...]
    m = jnp.mean(x, axis=1, keepdims=True)
    v = jnp.mean((x - m) ** 2, axis=1, keepdims=True)
    h = (x - m) * lax.rsqrt(v + 1e-5) * lng[...] + lnb[...]
    h = jnp.maximum(jnp.dot(h, w1[...], preferred_element_type=jnp.float32) + b1[...], 0.0)
    h = jnp.maximum(jnp.dot(h, w2[...], preferred_element_type=jnp.float32) + b2[...], 0.0)
    o_ref[...] = h


def enc(x, lng, lnb, w1, b1, w2, b2):
    return pl.pallas_call(
        _enc_block, grid=(GRID,),
        in_specs=[_rb(6), _full((6,)), _full((6,)), _full((6, 32)),
                  _full((32,)), _full((32, 32)), _full((32,))],
        out_specs=_rb(32),
        out_shape=jax.ShapeDtypeStruct((N, 32), jnp.float32),
    )(x, lng, lnb, w1, b1, w2, b2)


def _g_block(p0, p1, h0, w, o_dis, o_g0, o_g1):
    deg = p0[...] + p1[...] + 1.0
    dis = lax.rsqrt(deg)
    g = jnp.dot(h0[...], w[...], preferred_element_type=jnp.float32) * dis
    o_dis[...] = dis
    o_g0[...] = g[:, :16]
    o_g1[...] = g[:, 16:]


def gprep(p0, p1, h0, w):
    return pl.pallas_call(
        _g_block, grid=(GRID,),
        in_specs=[_rb(1), _rb(1), _rb(32), _full((32, 32))],
        out_specs=[_rb(1), _rb(16), _rb(16)],
        out_shape=[jax.ShapeDtypeStruct((N, 1), jnp.float32),
                   jax.ShapeDtypeStruct((N, 16), jnp.float32),
                   jax.ShapeDtypeStruct((N, 16), jnp.float32)],
    )(p0, p1, h0, w)


def zstat16x2(s0, s1, g0, g1, dis, bias):
    def blk(s0r, s1r, g0r, g1r, disr, br, o_z, o_st):
        d = disr[...]
        z = jnp.concatenate([d * (s0r[...] + g0r[...]),
                             d * (s1r[...] + g1r[...])], axis=1) + br[...]
        o_z[...] = z

        @pl.when(pl.program_id(0) == 0)
        def _():
            o_st[...] = jnp.zeros_like(o_st)

        o_st[...] += jnp.stack([jnp.sum(z, axis=0), jnp.sum(z * z, axis=0)])

    return pl.pallas_call(
        blk, grid=(GRID,),
        in_specs=[_rb(16), _rb(16), _rb(16), _rb(16), _rb(1), _full((32,))],
        out_specs=[_rb(32), _full((2, 32))],
        out_shape=[jax.ShapeDtypeStruct((N, 32), jnp.float32),
                   jax.ShapeDtypeStruct((2, 32), jnp.float32)],
    )(s0, s1, g0, g1, dis, bias)


def bn1_gat_prep(z, st, bng, bnb, gat_w, gat_as, gat_ad):
    def blk(zr, str_, bngr, bnbr, wr, asr, adr, o_h0, o_h1, o_sa, o_mx):
        mean = str_[0] / N
        var = str_[1] / N - mean * mean
        h = (zr[...] - mean) * lax.rsqrt(var + 1e-5) * bngr[...] + bnbr[...]
        h = jnp.maximum(h, 0.0)
        hh = jnp.dot(h, wr[...], preferred_element_type=jnp.float32)
        hh0 = hh[:, :16]
        hh1 = hh[:, 16:]
        a0 = jnp.sum(hh0 * asr[0], axis=1)
        a1 = jnp.sum(hh1 * asr[1], axis=1)
        d0 = jnp.sum(hh0 * adr[0], axis=1)
        d1 = jnp.sum(hh1 * adr[1], axis=1)
        o_h0[...] = hh0
        o_h1[...] = hh1
        o_sa[...] = jnp.stack([a0, a1, d0, d1], axis=1)

        @pl.when(pl.program_id(0) == 0)
        def _():
            o_mx[...] = jnp.full_like(o_mx, -3.0e38)

        mx = jnp.stack([jnp.max(a0), jnp.max(a1), jnp.max(d0), jnp.max(d1)])
        o_mx[...] = jnp.maximum(o_mx[...], mx[None, :])

    return pl.pallas_call(
        blk, grid=(GRID,),
        in_specs=[_rb(32), _full((2, 32)), _full((32,)), _full((32,)),
                  _full((32, 32)), _full((2, 16)), _full((2, 16))],
        out_specs=[_rb(16), _rb(16), _rb(4), _full((1, 4))],
        out_shape=[jax.ShapeDtypeStruct((N, 16), jnp.float32),
                   jax.ShapeDtypeStruct((N, 16), jnp.float32),
                   jax.ShapeDtypeStruct((N, 4), jnp.float32),
                   jax.ShapeDtypeStruct((1, 4), jnp.float32)],
    )(z, st, bng, bnb, gat_w, gat_as, gat_ad)


def gat_epilogue(on0, on1, od0, od1, hh0, hh1, sa, mrow, gat_b):
    def blk(on0r, on1r, od0r, od1r, h0r, h1r, sar, mr, br, o_z, o_st):
        es = sar[..., 0:2] + sar[..., 2:4]
        ws = jnp.exp(jnp.where(es > 0, es, 0.2 * es) - mr[...])
        n0 = on0r[...] + h0r[...] * ws[:, 0:1]
        n1 = on1r[...] + h1r[...] * ws[:, 1:2]
        d0 = od0r[...] + ws[:, 0:1]
        d1 = od1r[...] + ws[:, 1:2]
        z = jnp.concatenate([n0 / (d0 + 1e-16), n1 / (d1 + 1e-16)], axis=1) + br[...]
        o_z[...] = z

        @pl.when(pl.program_id(0) == 0)
        def _():
            o_st[...] = jnp.zeros_like(o_st)

        o_st[...] += jnp.stack([jnp.sum(z, axis=0), jnp.sum(z * z, axis=0)])

    return pl.pallas_call(
        blk, grid=(GRID,),
        in_specs=[_rb(16), _rb(16), _rb(1), _rb(1), _rb(16), _rb(16),
                  _rb(4), _full((1, 2)), _full((32,))],
        out_specs=[_rb(32), _full((2, 32))],
        out_shape=[jax.ShapeDtypeStruct((N, 32), jnp.float32),
                   jax.ShapeDtypeStruct((2, 32), jnp.float32)],
    )(on0, on1, od0, od1, hh0, hh1, sa, mrow, gat_b)


def bn2_f(z, st, bng, bnb, w2, dis):
    def blk(zr, str_, bngr, bnbr, wr, disr, o_f):
        mean = str_[0] / N
        var = str_[1] / N - mean * mean
        h = (zr[...] - mean) * lax.rsqrt(var + 1e-5) * bngr[...] + bnbr[...]
        h = jnp.maximum(h, 0.0)
        o_f[...] = jnp.dot(h, wr[...], preferred_element_type=jnp.float32) * disr[...]

    return pl.pallas_call(
        blk, grid=(GRID,),
        in_specs=[_rb(32), _full((2, 32)), _full((32,)), _full((32,)),
                  _full((32, 16)), _rb(1)],
        out_specs=_rb(16),
        out_shape=jax.ShapeDtypeStruct((N, 16), jnp.float32),
    )(z, st, bng, bnb, w2, dis)


def z3stat(q0, q1, f, dis, bias):
    def blk(q0r, q1r, fr, disr, br, o_z, o_st):
        z = disr[...] * (q0r[...] + q1r[...] + fr[...]) + br[...]
        o_z[...] = z

        @pl.when(pl.program_id(0) == 0)
        def _():
            o_st[...] = jnp.zeros_like(o_st)

        o_st[...] += jnp.stack([jnp.sum(z, axis=0), jnp.sum(z * z, axis=0)])

    return pl.pallas_call(
        blk, grid=(GRID,),
        in_specs=[_rb(16), _rb(16), _rb(16), _rb(1), _full((16,))],
        out_specs=[_rb(16), _full((2, 16))],
        out_shape=[jax.ShapeDtypeStruct((N, 16), jnp.float32),
                   jax.ShapeDtypeStruct((2, 16), jnp.float32)],
    )(q0, q1, f, dis, bias)


def final_head(z, st, bng, bnb, w1, b1, w2, b2):
    def blk(zr, str_, bngr, bnbr, w1r, b1r, w2r, b2r, o):
        mean = str_[0] / N
        var = str_[1] / N - mean * mean
        h = (zr[...] - mean) * lax.rsqrt(var + 1e-5) * bngr[...] + bnbr[...]
        h = jnp.maximum(h, 0.0)
        h = jnp.maximum(jnp.dot(h, w1r[...], preferred_element_type=jnp.float32) + b1r[...], 0.0)
        o[...] = jax.nn.sigmoid(jnp.dot(h, w2r[...], preferred_element_type=jnp.float32) + b2r[...])

    return pl.pallas_call(
        blk, grid=(GRID,),
        in_specs=[_rb(16), _full((2, 16)), _full((16,)), _full((16,)),
                  _full((16, 8)), _full((8,)), _full((8, 1)), _full((1,))],
        out_specs=_rb(1),
        out_shape=jax.ShapeDtypeStruct((N, 1), jnp.float32),
    )(z, st, bng, bnb, w1, b1, w2, b2)


def kernel(x, edge_index, ln_g, ln_b, enc_w1, enc_b1, enc_w2, enc_b2, gcn1_w, gcn1_b, bn1_g, bn1_b, gat_w, gat_as, gat_ad, gat_b, bn2_g, bn2_b, gcn2_w, gcn2_b, bn3_g, bn3_b, pr_w1, pr_b1, pr_w2, pr_b2):
    src = edge_index[0]
    dst = edge_index[1]
    src2d = jnp.concatenate(
        [src, jnp.zeros((EPAD,), jnp.int32)]).reshape(EROWS, 128)
    dst2d = jnp.concatenate(
        [dst, jnp.full((EPAD,), N, jnp.int32)]).reshape(EROWS, 128)

    h0 = enc(x, ln_g, ln_b, enc_w1, enc_b1, enc_w2, enc_b2)

    p0, p1 = _deg_call(dst2d)
    dis, g0, g1 = gprep(p0.reshape(N, 1), p1.reshape(N, 1), h0, gcn1_w)

    s0, s1 = _segsum_chsplit(src2d, dst2d, g0, g1)
    z1, st1 = zstat16x2(s0, s1, g0, g1, dis, gcn1_b)
    hh0, hh1, sa, mx = bn1_gat_prep(z1, st1, bn1_g, bn1_b, gat_w, gat_as, gat_ad)
    bound = mx[0, 0:2] + mx[0, 2:4]
    M = jnp.where(bound > 0, bound, 0.2 * bound)

    zpad8 = jnp.zeros((8,), jnp.float32)
    on0, on1, wout = _gat_call(
        src2d, dst2d, hh0, hh1,
        jnp.concatenate([sa[:, 0], zpad8]), jnp.concatenate([sa[:, 1], zpad8]),
        jnp.concatenate([sa[:, 2], zpad8]), jnp.concatenate([sa[:, 3], zpad8]),
        jnp.broadcast_to(M[:, None], (2, 16)))
    d0, d1 = _wden_call(wout, dst2d)
    z2, st2 = gat_epilogue(on0, on1, d0.reshape(N, 1), d1.reshape(N, 1),
                           hh0, hh1, sa, M.reshape(1, 2), gat_b)
    f = bn2_f(z2, st2, bn2_g, bn2_b, gcn2_w, dis)

    q0, q1 = _segsum_esplit(src2d, dst2d, f, f)
    z3, st3 = z3stat(q0, q1, f, dis, gcn2_b)
    out = final_head(z3, st3, bn3_g, bn3_b, pr_w1, pr_b1, pr_w2, pr_b2)
    return out.reshape(N)
